# Initial kernel scaffold; baseline (speedup 1.0000x reference)
import jax
import jax.numpy as jnp
from jax.experimental import pallas as pl


def kernel(x, edge_index, edge_weight, W1, W2, W3, W_mu, W_logvar):
    raise NotImplementedError("write your pallas kernel here")


# speedup vs baseline: 5.5332x; 5.5332x over previous
import functools

import jax
import jax.numpy as jnp
from jax import lax
from jax.experimental import pallas as pl
from jax.experimental.pallas import tpu as pltpu
from jax.experimental.pallas import tpu_sc as plsc

_NC = 2
_NS = 16
_NW = _NC * _NS
_CHUNK = 128
_LANES = 16


def _spmm_sc(support, src3, dst3, w3):
    fsplit = support.ndim == 3
    if fsplit:
        ncsup, n, feat = support.shape
        assert ncsup == _NC
    else:
        n, feat = support.shape
    ntiles_e, k_chunks, chunk = src3.shape
    assert chunk == _CHUNK and feat % _LANES == 0
    assert ntiles_e == (_NS if fsplit else _NW)
    npad = pl.cdiv(n, _NS * _CHUNK) * _NS * _CHUNK
    rows_per_sub = npad // _NS
    zc = rows_per_sub
    for d in range(1, rows_per_sub + 1):
        if rows_per_sub % d == 0 and rows_per_sub // d <= 128:
            zc = rows_per_sub // d
            break
    nz = rows_per_sub // zc
    jfeat = feat // _LANES

    mesh = plsc.VectorSubcoreMesh(
        core_axis_name="c", subcore_axis_name="s",
        num_cores=_NC, num_subcores=_NS)

    @functools.partial(
        pl.kernel,
        out_type=jax.ShapeDtypeStruct((_NC, npad, feat), jnp.float32),
        mesh=mesh,
        scratch_types=[
            pltpu.VMEM((k_chunks, _CHUNK), jnp.int32),
            pltpu.VMEM((k_chunks, _CHUNK), jnp.int32),
            pltpu.VMEM((k_chunks, _CHUNK), jnp.float32),
            pltpu.VMEM((_CHUNK, feat), jnp.float32),
            pltpu.VMEM((zc, feat), jnp.float32),
            pltpu.VMEM_SHARED((npad, feat), jnp.float32),
            pltpu.SemaphoreType.DMA,
        ],
        compiler_params=pltpu.CompilerParams(use_tc_tiling_on_sc=False),
    )
    def k(sup_hbm, src_hbm, dst_hbm, w_hbm, out_hbm,
          src_v, dst_v, w_v, rows_v, zbuf_v, acc_sh, sem):
        c = lax.axis_index("c")
        s = lax.axis_index("s")
        wid = s if fsplit else c * _NS + s
        zero = jnp.zeros((_LANES,), jnp.float32)

        def zrow(i, carry):
            for j in range(jfeat):
                zbuf_v[i, pl.ds(j * _LANES, _LANES)] = zero
            return carry
        lax.fori_loop(0, zc, zrow, 0)
        for b in range(nz):
            pltpu.sync_copy(zbuf_v, acc_sh.at[pl.ds(s * rows_per_sub + b * zc, zc)])

        pltpu.sync_copy(src_hbm.at[wid], src_v)
        pltpu.sync_copy(dst_hbm.at[wid], dst_v)
        pltpu.sync_copy(w_hbm.at[wid], w_v)
        plsc.subcore_barrier()

        table = sup_hbm.at[c] if fsplit else sup_hbm

        def chunk_body(kk, carry):
            pltpu.async_copy(table.at[src_v.at[kk]], rows_v, sem).wait()

            def group_body(g, inner):
                w16 = w_v[kk, pl.ds(g * _LANES, _LANES)]
                for i in range(_LANES):
                    wgt = w16[i]
                    e = g * _LANES + i
                    for j in range(jfeat):
                        sl = pl.ds(j * _LANES, _LANES)
                        rows_v[e, sl] = rows_v[e, sl] * wgt
                return inner
            lax.fori_loop(0, _CHUNK // _LANES, group_body, 0)
            pltpu.sync_copy(rows_v, acc_sh.at[dst_v.at[kk]], add=True)
            return carry
        lax.fori_loop(0, k_chunks, chunk_body, 0)

        plsc.subcore_barrier()
        sl = pl.ds(s * rows_per_sub, rows_per_sub)
        pltpu.sync_copy(acc_sh.at[sl], out_hbm.at[c, sl])

    return k(support, src3, dst3, w3)


_ROWS = 512


def _mm_tc(parts, Ws, relu_in):
    n, din = parts[0].shape
    grid = (pl.cdiv(n, _ROWS),)

    def body(*refs):
        nparts, nws = len(parts), len(Ws)
        p_refs = refs[:nparts]
        w_refs = refs[nparts:nparts + nws]
        o_refs = refs[nparts + nws:]
        h = p_refs[0][...]
        for pr in p_refs[1:]:
            h = h + pr[...]
        if relu_in:
            h = jnp.maximum(h, 0.0)
        for wr, orf in zip(w_refs, o_refs):
            orf[...] = jnp.dot(h, wr[...], preferred_element_type=jnp.float32)

    in_specs = (
        [pl.BlockSpec((_ROWS, din), lambda i: (i, 0)) for _ in parts]
        + [pl.BlockSpec((W.shape[0], W.shape[1]), lambda i: (0, 0)) for W in Ws])
    out_specs = [pl.BlockSpec((_ROWS, W.shape[1]), lambda i: (i, 0)) for W in Ws]
    outs = pl.pallas_call(
        body,
        grid=grid,
        in_specs=in_specs,
        out_specs=out_specs,
        out_shape=[jax.ShapeDtypeStruct((n, W.shape[1]), jnp.float32) for W in Ws],
    )(*parts, *Ws)
    return outs


def _mm_fsplit_tc(p_lo, p_hi, W_lo, W_hi, relu_in):
    n, dh = p_lo.shape
    dout = W_lo.shape[1]
    grid = (pl.cdiv(n, _ROWS),)

    def body(a_ref, b_ref, wl_ref, wh_ref, o_ref):
        a, b = a_ref[...], b_ref[...]
        if relu_in:
            a = jnp.maximum(a, 0.0)
            b = jnp.maximum(b, 0.0)
        o_ref[...] = (jnp.dot(a, wl_ref[...], preferred_element_type=jnp.float32)
                      + jnp.dot(b, wh_ref[...], preferred_element_type=jnp.float32))

    return pl.pallas_call(
        body,
        grid=grid,
        in_specs=[pl.BlockSpec((_ROWS, dh), lambda i: (i, 0))] * 2
        + [pl.BlockSpec((dh, dout), lambda i: (0, 0))] * 2,
        out_specs=pl.BlockSpec((_ROWS, dout), lambda i: (i, 0)),
        out_shape=jax.ShapeDtypeStruct((n, dout), jnp.float32),
    )(p_lo, p_hi, W_lo, W_hi)


def _relu_sum_tc(p0, p1):
    n, d = p0.shape
    grid = (pl.cdiv(n, _ROWS),)

    def body(a_ref, b_ref, o_ref):
        o_ref[...] = jnp.maximum(a_ref[...] + b_ref[...], 0.0)

    return pl.pallas_call(
        body,
        grid=grid,
        in_specs=[pl.BlockSpec((_ROWS, d), lambda i: (i, 0))] * 2,
        out_specs=pl.BlockSpec((_ROWS, d), lambda i: (i, 0)),
        out_shape=jax.ShapeDtypeStruct((n, d), jnp.float32),
    )(p0, p1)


def _chunk_edges(src, dst, w, ntiles):
    e = w.shape[0]
    ept = pl.cdiv(e, ntiles * _CHUNK) * _CHUNK
    pad = ept * ntiles - e
    src3 = jnp.pad(src, (0, pad)).reshape(ntiles, ept // _CHUNK, _CHUNK)
    dst3 = jnp.pad(dst, (0, pad)).reshape(ntiles, ept // _CHUNK, _CHUNK)
    w3 = jnp.pad(w, (0, pad)).reshape(ntiles, ept // _CHUNK, _CHUNK)
    return src3, dst3, w3


def kernel(x, edge_index, edge_weight, W1, W2, W3, W_mu, W_logvar):
    n = x.shape[0]
    src, dst = edge_index[0], edge_index[1]
    s16, d16, w16 = _chunk_edges(src, dst, edge_weight, _NS)
    s32, d32, w32 = _chunk_edges(src, dst, edge_weight, _NW)

    half = W1.shape[1] // 2
    s1_halves = _mm_tc([x], [W1[:, :half], W1[:, half:]], relu_in=False)
    sup1 = jnp.stack(s1_halves)
    p1 = _spmm_sc(sup1, s16, d16, w16)
    s2 = _mm_fsplit_tc(p1[0], p1[1], W2[:half], W2[half:], relu_in=True)
    p2 = _spmm_sc(s2, s32, d32, w32)
    s3 = _mm_tc([p2[0], p2[1]], [W3], relu_in=True)[0]
    p3 = _spmm_sc(s3, s32, d32, w32)
    h3 = _relu_sum_tc(p3[0], p3[1])
    p4 = _spmm_sc(h3, s32, d32, w32)
    mu, logvar = _mm_tc([p4[0], p4[1]], [W_mu, W_logvar], relu_in=False)
    mu, logvar = mu[:n], logvar[:n]
    return (mu, mu, logvar)


# speedup vs baseline: 5.5828x; 1.0090x over previous
import functools

import jax
import jax.numpy as jnp
from jax import lax
from jax.experimental import pallas as pl
from jax.experimental.pallas import tpu as pltpu
from jax.experimental.pallas import tpu_sc as plsc

_NC = 2
_NS = 16
_NW = _NC * _NS
_CHUNK = 128
_LANES = 16
_PBLK = 2


def _spmm_sc(support, src3, dst3, w3):
    fsplit = support.ndim == 3
    if fsplit:
        ncsup, n, feat = support.shape
        assert ncsup == _NC
    else:
        n, feat = support.shape
    ntiles_e, k_chunks, chunk = src3.shape
    assert chunk == _CHUNK and feat % _LANES == 0
    assert ntiles_e == (_NS if fsplit else _NW)
    npad = pl.cdiv(n, _NS * _CHUNK) * _NS * _CHUNK
    rows_per_sub = npad // _NS
    zc = rows_per_sub
    for d in range(1, rows_per_sub + 1):
        if rows_per_sub % d == 0 and rows_per_sub // d <= 128:
            zc = rows_per_sub // d
            break
    nz = rows_per_sub // zc
    jfeat = feat // _LANES

    mesh = plsc.VectorSubcoreMesh(
        core_axis_name="c", subcore_axis_name="s",
        num_cores=_NC, num_subcores=_NS)

    @functools.partial(
        pl.kernel,
        out_type=jax.ShapeDtypeStruct((_NC, npad, feat), jnp.float32),
        mesh=mesh,
        scratch_types=[
            pltpu.VMEM((k_chunks, _CHUNK), jnp.int32),
            pltpu.VMEM((k_chunks, _CHUNK), jnp.int32),
            pltpu.VMEM((k_chunks, _CHUNK), jnp.float32),
            pltpu.VMEM((_CHUNK, feat), jnp.float32),
            pltpu.VMEM((zc, feat), jnp.float32),
            pltpu.VMEM_SHARED((npad, feat), jnp.float32),
            pltpu.SemaphoreType.DMA,
        ],
        compiler_params=pltpu.CompilerParams(use_tc_tiling_on_sc=False),
    )
    def k(sup_hbm, src_hbm, dst_hbm, w_hbm, out_hbm,
          src_v, dst_v, w_v, rows_v, zbuf_v, acc_sh, sem):
        c = lax.axis_index("c")
        s = lax.axis_index("s")
        wid = s if fsplit else c * _NS + s
        zero = jnp.zeros((_LANES,), jnp.float32)

        def zrow(i, carry):
            for j in range(jfeat):
                zbuf_v[i, pl.ds(j * _LANES, _LANES)] = zero
            return carry
        lax.fori_loop(0, zc, zrow, 0)
        for b in range(nz):
            pltpu.sync_copy(zbuf_v, acc_sh.at[pl.ds(s * rows_per_sub + b * zc, zc)])

        pltpu.sync_copy(src_hbm.at[wid], src_v)
        pltpu.sync_copy(dst_hbm.at[wid], dst_v)
        pltpu.sync_copy(w_hbm.at[wid], w_v)
        plsc.subcore_barrier()

        table = sup_hbm.at[c] if fsplit else sup_hbm

        def chunk_body(kk, carry):
            pltpu.async_copy(table.at[src_v.at[kk]], rows_v, sem).wait()

            def group_body(g, inner):
                w16 = w_v[kk, pl.ds(g * _LANES, _LANES)]
                for i in range(_LANES):
                    wgt = w16[i]
                    e = g * _LANES + i
                    for j in range(jfeat):
                        sl = pl.ds(j * _LANES, _LANES)
                        rows_v[e, sl] = rows_v[e, sl] * wgt
                return inner
            lax.fori_loop(0, _CHUNK // _LANES, group_body, 0)
            pltpu.sync_copy(rows_v, acc_sh.at[dst_v.at[kk]], add=True)
            return carry
        lax.fori_loop(0, k_chunks, chunk_body, 0)

        plsc.subcore_barrier()
        sl = pl.ds(s * rows_per_sub, rows_per_sub)
        pltpu.sync_copy(acc_sh.at[sl], out_hbm.at[c, sl])

    return k(support, src3, dst3, w3)


def _spmm_sc_pipe(support, src3, dst3, w3):
    fsplit = support.ndim == 3
    if fsplit:
        ncsup, n, feat = support.shape
        assert ncsup == _NC
    else:
        n, feat = support.shape
    ntiles_e, k_chunks, chunk = src3.shape
    assert chunk == _CHUNK and feat % _LANES == 0
    assert ntiles_e == (_NS if fsplit else _NW)
    assert k_chunks % (2 * _PBLK) == 0
    nb = k_chunks // _PBLK
    nb2 = nb // 2
    blk_e = _PBLK * _CHUNK
    npad = pl.cdiv(n, _NS * _CHUNK) * _NS * _CHUNK
    rows_per_sub = npad // _NS
    jfeat = feat // _LANES

    mesh = plsc.VectorSubcoreMesh(
        core_axis_name="c", subcore_axis_name="s",
        num_cores=_NC, num_subcores=_NS)

    @functools.partial(
        pl.kernel,
        out_type=jax.ShapeDtypeStruct((_NC, npad, feat), jnp.float32),
        mesh=mesh,
        scratch_types=[
            pltpu.VMEM((k_chunks, _CHUNK), jnp.int32),
            pltpu.VMEM((k_chunks, _CHUNK), jnp.int32),
            pltpu.VMEM((k_chunks * _CHUNK,), jnp.float32),
            pltpu.VMEM((2, blk_e, feat), jnp.float32),
            pltpu.VMEM_SHARED((npad, feat), jnp.float32),
            pltpu.SemaphoreType.DMA,
            pltpu.SemaphoreType.DMA,
            pltpu.SemaphoreType.DMA,
            pltpu.SemaphoreType.DMA,
        ],
        compiler_params=pltpu.CompilerParams(use_tc_tiling_on_sc=False),
    )
    def k(sup_hbm, src_hbm, dst_hbm, w_hbm, out_hbm,
          src_v, dst_v, w_v, rows2, acc_sh, sg0, sg1, ss0, ss1):
        c = lax.axis_index("c")
        s = lax.axis_index("s")
        wid = s if fsplit else c * _NS + s
        zero = jnp.zeros((_LANES,), jnp.float32)
        rows0 = rows2.at[0]
        rows1 = rows2.at[1]
        table = sup_hbm.at[c] if fsplit else sup_hbm
        dummy = table.at[pl.ds(0, blk_e)]

        def zrow(i, carry):
            for j in range(jfeat):
                rows0[i, pl.ds(j * _LANES, _LANES)] = zero
            return carry
        lax.fori_loop(0, blk_e, zrow, 0)
        offs = 0
        while offs < rows_per_sub:
            sz = min(blk_e, rows_per_sub - offs)
            pltpu.sync_copy(rows2.at[0, pl.ds(0, sz)],
                            acc_sh.at[pl.ds(s * rows_per_sub + offs, sz)])
            offs += sz

        pltpu.sync_copy(src_hbm.at[wid], src_v)
        pltpu.sync_copy(dst_hbm.at[wid], dst_v)
        pltpu.sync_copy(w_hbm.at[wid], w_v)
        plsc.subcore_barrier()

        def issue_gathers(t, rows_p, sem):
            for b in range(_PBLK):
                pltpu.async_copy(table.at[src_v.at[t * _PBLK + b]],
                                 rows_p.at[pl.ds(b * _CHUNK, _CHUNK)], sem)

        def issue_scatters(t, rows_p, sem):
            del sem
            for b in range(_PBLK):
                pltpu.sync_copy(rows_p.at[pl.ds(b * _CHUNK, _CHUNK)],
                                acc_sh.at[dst_v.at[t * _PBLK + b]], add=True)

        def drain(sem, ref):
            pltpu.make_async_copy(dummy, ref, sem).wait()

        def scale(t, rows_p):
            def gbody(g, carry):
                w16 = w_v[pl.ds(t * blk_e + g * _LANES, _LANES)]
                for i in range(_LANES):
                    wgt = w16[i]
                    e = g * _LANES + i
                    for j in range(jfeat):
                        sl = pl.ds(j * _LANES, _LANES)
                        rows_p[e, sl] = rows_p[e, sl] * wgt
                return carry
            lax.fori_loop(0, blk_e // _LANES, gbody, 0)

        issue_gathers(0, rows0, sg0)

        def pair(t2, carry):
            t0 = t2 * 2

            issue_gathers(t0 + 1, rows1, sg1)
            drain(sg0, rows0)
            scale(t0, rows0)
            issue_scatters(t0, rows0, ss0)

            @pl.when(t2 < nb2 - 1)
            def _refill0():
                issue_gathers(t0 + 2, rows0, sg0)
            drain(sg1, rows1)
            scale(t0 + 1, rows1)
            issue_scatters(t0 + 1, rows1, ss1)
            return carry
        lax.fori_loop(0, nb2, pair, 0)

        plsc.subcore_barrier()
        sl = pl.ds(s * rows_per_sub, rows_per_sub)
        pltpu.sync_copy(acc_sh.at[sl], out_hbm.at[c, sl])

    return k(support, src3, dst3, w3)


_ROWS = 512


def _mm_tc(parts, Ws, relu_in):
    n, din = parts[0].shape
    grid = (pl.cdiv(n, _ROWS),)

    def body(*refs):
        nparts, nws = len(parts), len(Ws)
        p_refs = refs[:nparts]
        w_refs = refs[nparts:nparts + nws]
        o_refs = refs[nparts + nws:]
        h = p_refs[0][...]
        for pr in p_refs[1:]:
            h = h + pr[...]
        if relu_in:
            h = jnp.maximum(h, 0.0)
        for wr, orf in zip(w_refs, o_refs):
            orf[...] = jnp.dot(h, wr[...], preferred_element_type=jnp.float32)

    in_specs = (
        [pl.BlockSpec((_ROWS, din), lambda i: (i, 0)) for _ in parts]
        + [pl.BlockSpec((W.shape[0], W.shape[1]), lambda i: (0, 0)) for W in Ws])
    out_specs = [pl.BlockSpec((_ROWS, W.shape[1]), lambda i: (i, 0)) for W in Ws]
    outs = pl.pallas_call(
        body,
        grid=grid,
        in_specs=in_specs,
        out_specs=out_specs,
        out_shape=[jax.ShapeDtypeStruct((n, W.shape[1]), jnp.float32) for W in Ws],
    )(*parts, *Ws)
    return outs


def _mm_fsplit_tc(p_lo, p_hi, W_lo, W_hi, relu_in):
    n, dh = p_lo.shape
    dout = W_lo.shape[1]
    grid = (pl.cdiv(n, _ROWS),)

    def body(a_ref, b_ref, wl_ref, wh_ref, o_ref):
        a, b = a_ref[...], b_ref[...]
        if relu_in:
            a = jnp.maximum(a, 0.0)
            b = jnp.maximum(b, 0.0)
        o_ref[...] = (jnp.dot(a, wl_ref[...], preferred_element_type=jnp.float32)
                      + jnp.dot(b, wh_ref[...], preferred_element_type=jnp.float32))

    return pl.pallas_call(
        body,
        grid=grid,
        in_specs=[pl.BlockSpec((_ROWS, dh), lambda i: (i, 0))] * 2
        + [pl.BlockSpec((dh, dout), lambda i: (0, 0))] * 2,
        out_specs=pl.BlockSpec((_ROWS, dout), lambda i: (i, 0)),
        out_shape=jax.ShapeDtypeStruct((n, dout), jnp.float32),
    )(p_lo, p_hi, W_lo, W_hi)


def _relu_sum_tc(p0, p1):
    n, d = p0.shape
    grid = (pl.cdiv(n, _ROWS),)

    def body(a_ref, b_ref, o_ref):
        o_ref[...] = jnp.maximum(a_ref[...] + b_ref[...], 0.0)

    return pl.pallas_call(
        body,
        grid=grid,
        in_specs=[pl.BlockSpec((_ROWS, d), lambda i: (i, 0))] * 2,
        out_specs=pl.BlockSpec((_ROWS, d), lambda i: (i, 0)),
        out_shape=jax.ShapeDtypeStruct((n, d), jnp.float32),
    )(p0, p1)


def _chunk_edges(src, dst, w, ntiles, align=_CHUNK):
    e = w.shape[0]
    ept = pl.cdiv(e, ntiles * align) * align
    pad = ept * ntiles - e
    src3 = jnp.pad(src, (0, pad)).reshape(ntiles, ept // _CHUNK, _CHUNK)
    dst3 = jnp.pad(dst, (0, pad)).reshape(ntiles, ept // _CHUNK, _CHUNK)
    w3 = jnp.pad(w, (0, pad)).reshape(ntiles, ept // _CHUNK, _CHUNK)
    return src3, dst3, w3


def kernel(x, edge_index, edge_weight, W1, W2, W3, W_mu, W_logvar):
    n = x.shape[0]
    src, dst = edge_index[0], edge_index[1]
    s16, d16, w16 = _chunk_edges(src, dst, edge_weight, _NS)
    s32, d32, w32 = _chunk_edges(src, dst, edge_weight, _NW)

    half = W1.shape[1] // 2
    s1_halves = _mm_tc([x], [W1[:, :half], W1[:, half:]], relu_in=False)
    sup1 = jnp.stack(s1_halves)
    p1 = _spmm_sc(sup1, s16, d16, w16)
    s2 = _mm_fsplit_tc(p1[0], p1[1], W2[:half], W2[half:], relu_in=True)
    p2 = _spmm_sc(s2, s32, d32, w32)
    s3 = _mm_tc([p2[0], p2[1]], [W3], relu_in=True)[0]
    s32p, d32p, w32p = _chunk_edges(src, dst, edge_weight, _NW,
                                    align=2 * _PBLK * _CHUNK)
    w32f = w32p.reshape(_NW, -1)
    p3 = _spmm_sc_pipe(s3, s32p, d32p, w32f)
    h3 = _relu_sum_tc(p3[0], p3[1])
    p4 = _spmm_sc_pipe(h3, s32p, d32p, w32f)
    mu, logvar = _mm_tc([p4[0], p4[1]], [W_mu, W_logvar], relu_in=False)
    mu, logvar = mu[:n], logvar[:n]
    return (mu, mu, logvar)


# speedup vs baseline: 6.5802x; 1.1787x over previous
import functools

import jax
import jax.numpy as jnp
from jax import lax
from jax.experimental import pallas as pl
from jax.experimental.pallas import tpu as pltpu
from jax.experimental.pallas import tpu_sc as plsc

_NC = 2
_NS = 16
_NW = _NC * _NS
_CHUNK = 128
_LANES = 16
_PBLK = 2


def _spmm_sc(support, src3, dst3, w3):
    fsplit = support.ndim == 3
    if fsplit:
        ncsup, n, feat = support.shape
        assert ncsup == _NC
    else:
        n, feat = support.shape
    ntiles_e, k_chunks, chunk = src3.shape
    assert chunk == _CHUNK and feat % _LANES == 0
    assert ntiles_e == (_NS if fsplit else _NW)
    npad = pl.cdiv(n, _NS * _CHUNK) * _NS * _CHUNK
    rows_per_sub = npad // _NS
    zc = rows_per_sub
    for d in range(1, rows_per_sub + 1):
        if rows_per_sub % d == 0 and rows_per_sub // d <= 128:
            zc = rows_per_sub // d
            break
    nz = rows_per_sub // zc
    jfeat = feat // _LANES

    mesh = plsc.VectorSubcoreMesh(
        core_axis_name="c", subcore_axis_name="s",
        num_cores=_NC, num_subcores=_NS)

    @functools.partial(
        pl.kernel,
        out_type=jax.ShapeDtypeStruct((_NC, npad, feat), jnp.float32),
        mesh=mesh,
        scratch_types=[
            pltpu.VMEM((k_chunks, _CHUNK), jnp.int32),
            pltpu.VMEM((k_chunks, _CHUNK), jnp.int32),
            pltpu.VMEM((k_chunks, _CHUNK), jnp.float32),
            pltpu.VMEM((_CHUNK, feat), jnp.float32),
            pltpu.VMEM((zc, feat), jnp.float32),
            pltpu.VMEM_SHARED((npad, feat), jnp.float32),
            pltpu.SemaphoreType.DMA,
        ],
        compiler_params=pltpu.CompilerParams(use_tc_tiling_on_sc=False),
    )
    def k(sup_hbm, src_hbm, dst_hbm, w_hbm, out_hbm,
          src_v, dst_v, w_v, rows_v, zbuf_v, acc_sh, sem):
        c = lax.axis_index("c")
        s = lax.axis_index("s")
        wid = s if fsplit else c * _NS + s
        zero = jnp.zeros((_LANES,), jnp.float32)

        def zrow(i, carry):
            for j in range(jfeat):
                zbuf_v[i, pl.ds(j * _LANES, _LANES)] = zero
            return carry
        lax.fori_loop(0, zc, zrow, 0)
        for b in range(nz):
            pltpu.sync_copy(zbuf_v, acc_sh.at[pl.ds(s * rows_per_sub + b * zc, zc)])

        pltpu.sync_copy(src_hbm.at[wid], src_v)
        pltpu.sync_copy(dst_hbm.at[wid], dst_v)
        pltpu.sync_copy(w_hbm.at[wid], w_v)
        plsc.subcore_barrier()

        table = sup_hbm.at[c] if fsplit else sup_hbm

        def chunk_body(kk, carry):
            pltpu.async_copy(table.at[src_v.at[kk]], rows_v, sem).wait()

            def group_body(g, inner):
                w16 = w_v[kk, pl.ds(g * _LANES, _LANES)]
                for i in range(_LANES):
                    wgt = w16[i]
                    e = g * _LANES + i
                    for j in range(jfeat):
                        sl = pl.ds(j * _LANES, _LANES)
                        rows_v[e, sl] = rows_v[e, sl] * wgt
                return inner
            lax.fori_loop(0, _CHUNK // _LANES, group_body, 0)
            pltpu.sync_copy(rows_v, acc_sh.at[dst_v.at[kk]], add=True)
            return carry
        lax.fori_loop(0, k_chunks, chunk_body, 0)

        plsc.subcore_barrier()
        sl = pl.ds(s * rows_per_sub, rows_per_sub)
        pltpu.sync_copy(acc_sh.at[sl], out_hbm.at[c, sl])

    return k(support, src3, dst3, w3)


def _spmm_sc_pipe(support, src3, dst3, w3, pblk=_PBLK):
    fsplit = support.ndim == 3
    if fsplit:
        ncsup, n, feat = support.shape
        assert ncsup == _NC
    else:
        n, feat = support.shape
    ntiles_e, k_chunks, chunk = src3.shape
    assert chunk == _CHUNK and feat % _LANES == 0
    assert ntiles_e == (_NS if fsplit else _NW)
    assert k_chunks % (2 * pblk) == 0
    nb = k_chunks // pblk
    nb2 = nb // 2
    blk_e = pblk * _CHUNK
    npad = pl.cdiv(n, _NS * _CHUNK) * _NS * _CHUNK
    rows_per_sub = npad // _NS
    jfeat = feat // _LANES

    mesh = plsc.VectorSubcoreMesh(
        core_axis_name="c", subcore_axis_name="s",
        num_cores=_NC, num_subcores=_NS)

    @functools.partial(
        pl.kernel,
        out_type=jax.ShapeDtypeStruct((_NC, npad, feat), jnp.float32),
        mesh=mesh,
        scratch_types=[
            pltpu.VMEM((k_chunks, _CHUNK), jnp.int32),
            pltpu.VMEM((k_chunks, _CHUNK), jnp.int32),
            pltpu.VMEM((k_chunks * _CHUNK,), jnp.float32),
            pltpu.VMEM((2, blk_e, feat), jnp.float32),
            pltpu.VMEM_SHARED((npad, feat), jnp.float32),
            pltpu.SemaphoreType.DMA,
            pltpu.SemaphoreType.DMA,
            pltpu.SemaphoreType.DMA,
            pltpu.SemaphoreType.DMA,
        ],
        compiler_params=pltpu.CompilerParams(use_tc_tiling_on_sc=False),
    )
    def k(sup_hbm, src_hbm, dst_hbm, w_hbm, out_hbm,
          src_v, dst_v, w_v, rows2, acc_sh, sg0, sg1, ss0, ss1):
        c = lax.axis_index("c")
        s = lax.axis_index("s")
        wid = s if fsplit else c * _NS + s
        zero = jnp.zeros((_LANES,), jnp.float32)
        rows0 = rows2.at[0]
        rows1 = rows2.at[1]
        table = sup_hbm.at[c] if fsplit else sup_hbm
        dummy = table.at[pl.ds(0, blk_e)]

        def zrow(i, carry):
            for j in range(jfeat):
                rows0[i, pl.ds(j * _LANES, _LANES)] = zero
            return carry
        lax.fori_loop(0, blk_e, zrow, 0)
        offs = 0
        while offs < rows_per_sub:
            sz = min(blk_e, rows_per_sub - offs)
            pltpu.sync_copy(rows2.at[0, pl.ds(0, sz)],
                            acc_sh.at[pl.ds(s * rows_per_sub + offs, sz)])
            offs += sz

        pltpu.sync_copy(src_hbm.at[wid], src_v)
        pltpu.sync_copy(dst_hbm.at[wid], dst_v)
        pltpu.sync_copy(w_hbm.at[wid], w_v)
        plsc.subcore_barrier()

        def issue_gathers(t, rows_p, sem):
            for b in range(pblk):
                pltpu.async_copy(table.at[src_v.at[t * pblk + b]],
                                 rows_p.at[pl.ds(b * _CHUNK, _CHUNK)], sem)

        def issue_scatters(t, rows_p, sem):
            del sem
            for b in range(pblk):
                pltpu.sync_copy(rows_p.at[pl.ds(b * _CHUNK, _CHUNK)],
                                acc_sh.at[dst_v.at[t * pblk + b]], add=True)

        def drain(sem, ref):
            pltpu.make_async_copy(dummy, ref, sem).wait()

        def scale(t, rows_p):
            def gbody(g, carry):
                w16 = w_v[pl.ds(t * blk_e + g * _LANES, _LANES)]
                for i in range(_LANES):
                    wgt = w16[i]
                    e = g * _LANES + i
                    for j in range(jfeat):
                        sl = pl.ds(j * _LANES, _LANES)
                        rows_p[e, sl] = rows_p[e, sl] * wgt
                return carry
            lax.fori_loop(0, blk_e // _LANES, gbody, 0)

        issue_gathers(0, rows0, sg0)

        def pair(t2, carry):
            t0 = t2 * 2

            issue_gathers(t0 + 1, rows1, sg1)
            drain(sg0, rows0)
            scale(t0, rows0)
            issue_scatters(t0, rows0, ss0)

            @pl.when(t2 < nb2 - 1)
            def _refill0():
                issue_gathers(t0 + 2, rows0, sg0)
            drain(sg1, rows1)
            scale(t0 + 1, rows1)
            issue_scatters(t0 + 1, rows1, ss1)
            return carry
        lax.fori_loop(0, nb2, pair, 0)

        plsc.subcore_barrier()
        sl = pl.ds(s * rows_per_sub, rows_per_sub)
        pltpu.sync_copy(acc_sh.at[sl], out_hbm.at[c, sl])

    return k(support, src3, dst3, w3)


_ROWS = 512


def _mm_tc(parts, Ws, relu_in):
    n, din = parts[0].shape
    grid = (pl.cdiv(n, _ROWS),)

    def body(*refs):
        nparts, nws = len(parts), len(Ws)
        p_refs = refs[:nparts]
        w_refs = refs[nparts:nparts + nws]
        o_refs = refs[nparts + nws:]
        h = p_refs[0][...]
        for pr in p_refs[1:]:
            h = h + pr[...]
        if relu_in:
            h = jnp.maximum(h, 0.0)
        for wr, orf in zip(w_refs, o_refs):
            orf[...] = jnp.dot(h, wr[...], preferred_element_type=jnp.float32)

    in_specs = (
        [pl.BlockSpec((_ROWS, din), lambda i: (i, 0)) for _ in parts]
        + [pl.BlockSpec((W.shape[0], W.shape[1]), lambda i: (0, 0)) for W in Ws])
    out_specs = [pl.BlockSpec((_ROWS, W.shape[1]), lambda i: (i, 0)) for W in Ws]
    outs = pl.pallas_call(
        body,
        grid=grid,
        in_specs=in_specs,
        out_specs=out_specs,
        out_shape=[jax.ShapeDtypeStruct((n, W.shape[1]), jnp.float32) for W in Ws],
    )(*parts, *Ws)
    return outs


def _mm_fsplit_tc(p_lo, p_hi, W_lo, W_hi, relu_in):
    n, dh = p_lo.shape
    dout = W_lo.shape[1]
    grid = (pl.cdiv(n, _ROWS),)

    def body(a_ref, b_ref, wl_ref, wh_ref, o_ref):
        a, b = a_ref[...], b_ref[...]
        if relu_in:
            a = jnp.maximum(a, 0.0)
            b = jnp.maximum(b, 0.0)
        o_ref[...] = (jnp.dot(a, wl_ref[...], preferred_element_type=jnp.float32)
                      + jnp.dot(b, wh_ref[...], preferred_element_type=jnp.float32))

    return pl.pallas_call(
        body,
        grid=grid,
        in_specs=[pl.BlockSpec((_ROWS, dh), lambda i: (i, 0))] * 2
        + [pl.BlockSpec((dh, dout), lambda i: (0, 0))] * 2,
        out_specs=pl.BlockSpec((_ROWS, dout), lambda i: (i, 0)),
        out_shape=jax.ShapeDtypeStruct((n, dout), jnp.float32),
    )(p_lo, p_hi, W_lo, W_hi)


def _relu_sum_tc(p0, p1):
    n, d = p0.shape
    grid = (pl.cdiv(n, _ROWS),)

    def body(a_ref, b_ref, o_ref):
        o_ref[...] = jnp.maximum(a_ref[...] + b_ref[...], 0.0)

    return pl.pallas_call(
        body,
        grid=grid,
        in_specs=[pl.BlockSpec((_ROWS, d), lambda i: (i, 0))] * 2,
        out_specs=pl.BlockSpec((_ROWS, d), lambda i: (i, 0)),
        out_shape=jax.ShapeDtypeStruct((n, d), jnp.float32),
    )(p0, p1)


def _chunk_edges(src, dst, w, ntiles, align=_CHUNK):
    e = w.shape[0]
    ept = pl.cdiv(e, ntiles * align) * align
    pad = ept * ntiles - e
    src3 = jnp.pad(src, (0, pad)).reshape(ntiles, ept // _CHUNK, _CHUNK)
    dst3 = jnp.pad(dst, (0, pad)).reshape(ntiles, ept // _CHUNK, _CHUNK)
    w3 = jnp.pad(w, (0, pad)).reshape(ntiles, ept // _CHUNK, _CHUNK)
    return src3, dst3, w3


def kernel(x, edge_index, edge_weight, W1, W2, W3, W_mu, W_logvar):
    n = x.shape[0]
    src, dst = edge_index[0], edge_index[1]
    s16, d16, w16 = _chunk_edges(src, dst, edge_weight, _NS)
    s32, d32, w32 = _chunk_edges(src, dst, edge_weight, _NW)

    s16p, d16p, w16p = _chunk_edges(src, dst, edge_weight, _NS,
                                    align=2 * _CHUNK)
    s32q, d32q, w32q = _chunk_edges(src, dst, edge_weight, _NW,
                                    align=2 * _CHUNK)
    half = W1.shape[1] // 2
    s1_halves = _mm_tc([x], [W1[:, :half], W1[:, half:]], relu_in=False)
    sup1 = jnp.stack(s1_halves)
    p1 = _spmm_sc_pipe(sup1, s16p, d16p, w16p.reshape(_NS, -1), pblk=1)
    s2 = _mm_fsplit_tc(p1[0], p1[1], W2[:half], W2[half:], relu_in=True)
    p2 = _spmm_sc_pipe(s2, s32q, d32q, w32q.reshape(_NW, -1), pblk=1)
    s3 = _mm_tc([p2[0], p2[1]], [W3], relu_in=True)[0]
    s32p, d32p, w32p = _chunk_edges(src, dst, edge_weight, _NW,
                                    align=2 * _PBLK * _CHUNK)
    w32f = w32p.reshape(_NW, -1)
    p3 = _spmm_sc_pipe(s3, s32p, d32p, w32f)
    h3 = _relu_sum_tc(p3[0], p3[1])
    p4 = _spmm_sc_pipe(h3, s32p, d32p, w32f)
    mu, logvar = _mm_tc([p4[0], p4[1]], [W_mu, W_logvar], relu_in=False)
    mu, logvar = mu[:n], logvar[:n]
    return (mu, mu, logvar)
